# Initial kernel scaffold; baseline (speedup 1.0000x reference)
import jax
import jax.numpy as jnp
from jax.experimental import pallas as pl


def kernel(z_src, z_dst, lr_pair_idx, W_proj, b_proj, W_bil, b_bil):
    raise NotImplementedError("write your pallas kernel here")


# speedup vs baseline: 1.2024x; 1.2024x over previous
import functools

import jax
import jax.numpy as jnp
from jax import lax
from jax.experimental import pallas as pl
from jax.experimental.pallas import tpu as pltpu

D = 768
P = 8
E = 4096
T = 256
NT = E // T
W = NT + P - 1


def _grouped_kernel(sched_ref, offs_ref,
                    zsrc_ref, zdst_ref, wp_ref, wb_ref, bp_ref, bbil_ref,
                    out_ref):
    w = pl.program_id(0)
    expert = sched_ref[1, w]
    first = sched_ref[2, w]
    valid = sched_ref[3, w]
    tile = sched_ref[0, w]

    @pl.when(first == 1)
    def _init():
        out_ref[...] = jnp.zeros_like(out_ref)

    @pl.when(valid == 1)
    def _body():
        zsrc = zsrc_ref[...]
        zdst = zdst_ref[...]
        wp = wp_ref[0]
        wb = wb_ref[0]
        bp = bp_ref[0]
        u = lax.dot_general(zdst, wb, (((1,), (1,)), ((), ())),
                            preferred_element_type=jnp.float32)
        proj = lax.dot_general(zsrc, wp, (((1,), (1,)), ((), ())),
                               preferred_element_type=jnp.float32) + bp
        s = jnp.sum(proj * u, axis=1, keepdims=True) + bbil_ref[0, 0]
        rows = tile * T + lax.broadcasted_iota(jnp.int32, (T, 1), 0)
        lo = offs_ref[expert]
        hi = offs_ref[expert + 1]
        mask = (rows >= lo) & (rows < hi)
        out_ref[...] += jnp.where(mask, s, 0.0)


def _grouped_scores(sched, offs, z_src_perm, z_dst_perm, W_proj, b_proj,
                    W_bil, b_bil):
    bp3 = b_proj.reshape(P, 1, D)
    bb2 = b_bil.reshape(1, 1)
    grid_spec = pltpu.PrefetchScalarGridSpec(
        num_scalar_prefetch=2,
        grid=(W,),
        in_specs=[
            pl.BlockSpec((T, D), lambda w, s, o: (s[0, w], 0)),
            pl.BlockSpec((T, D), lambda w, s, o: (s[0, w], 0)),
            pl.BlockSpec((1, D, D), lambda w, s, o: (s[1, w], 0, 0)),
            pl.BlockSpec((1, D, D), lambda w, s, o: (0, 0, 0)),
            pl.BlockSpec((1, 1, D), lambda w, s, o: (s[1, w], 0, 0)),
            pl.BlockSpec((1, 1), lambda w, s, o: (0, 0)),
        ],
        out_specs=pl.BlockSpec((T, 1), lambda w, s, o: (s[0, w], 0)),
    )
    return pl.pallas_call(
        _grouped_kernel,
        grid_spec=grid_spec,
        out_shape=jax.ShapeDtypeStruct((E, 1), jnp.float32),
    )(sched, offs, z_src_perm, z_dst_perm, W_proj, W_bil, bp3, bb2)


def _build_schedule(offs):
    tt = jnp.arange(NT, dtype=jnp.int32)[:, None]
    ee = jnp.arange(P, dtype=jnp.int32)[None, :]
    lo = jnp.maximum(offs[:-1][None, :], tt * T)
    hi = jnp.minimum(offs[1:][None, :], (tt + 1) * T)
    valid = (hi > lo).reshape(-1)
    order = jnp.argsort(~valid, stable=True)
    sel = order[:W]
    vsel = valid[sel].astype(jnp.int32)
    tile = jnp.where(vsel == 1, (sel // P).astype(jnp.int32), NT - 1)
    expert = jnp.where(vsel == 1, (sel % P).astype(jnp.int32), P - 1)
    prev_tile = jnp.concatenate([jnp.array([-1], jnp.int32), tile[:-1]])
    first = ((tile != prev_tile) & (vsel == 1)).astype(jnp.int32)
    return jnp.stack([tile, expert, first, vsel])


def kernel(z_src, z_dst, lr_pair_idx, W_proj, b_proj, W_bil, b_bil):
    idx = lr_pair_idx.astype(jnp.int32)
    perm = jnp.argsort(idx, stable=True)
    z_src_perm = jnp.take(z_src, perm, axis=0)
    z_dst_perm = jnp.take(z_dst, perm, axis=0)
    counts = jnp.sum(idx[None, :] == jnp.arange(P, dtype=jnp.int32)[:, None],
                     axis=1, dtype=jnp.int32)
    offs = jnp.concatenate([jnp.zeros((1,), jnp.int32), jnp.cumsum(counts)])
    offs = offs.astype(jnp.int32)
    sched = _build_schedule(offs)
    scores_perm = _grouped_scores(sched, offs, z_src_perm, z_dst_perm,
                                  W_proj, b_proj, W_bil, b_bil)
    scores = jnp.zeros((E, 1), jnp.float32).at[perm].set(scores_perm)
    return scores
